# baseline (speedup 1.0000x reference)
import functools

import jax
import jax.numpy as jnp
from jax import lax
from jax.experimental import pallas as pl
from jax.experimental.pallas import tpu as pltpu
from jax.experimental.pallas import tpu_sc as plsc

N_NODES = 10000
N_EDGES = 320000
D = 128

NUM_TILES = 32
CHUNK = 128
CHUNKS_PER_TILE = 79
E_PAD = NUM_TILES * CHUNKS_PER_TILE * CHUNK
ACC_ROWS = 10240
ROWS_PER_TILE = ACC_ROWS // 16


def _prep_edges(edge_index):
    ei = edge_index.astype(jnp.int32).reshape(2, N_EDGES // CHUNK, CHUNK)
    npad = E_PAD - N_EDGES
    ar = jnp.arange(npad, dtype=jnp.int32)
    pad_src = ((ar * 97) % N_NODES).reshape(npad // CHUNK, CHUNK)
    pad_dst = (N_NODES + (ar % (ACC_ROWS - N_NODES))).reshape(npad // CHUNK, CHUNK)
    srcb = jnp.concatenate([ei[0], pad_src]).reshape(NUM_TILES, CHUNKS_PER_TILE, 1, CHUNK)
    dstb = jnp.concatenate([ei[1], pad_dst]).reshape(NUM_TILES, CHUNKS_PER_TILE, CHUNK)
    return srcb, dstb


def _sc_agg_body(with_deg, x_hbm, srcb_hbm, dstb_hbm, out_hbm, outdeg_hbm,
                 src_v, dst_v, buf0, buf1, ones_v, zb_v, acc_sh, deg_sh,
                 isem, sl0, sl1, gsem0, gsem1, ssem0, ssem1, dsem):
    cid = lax.axis_index("c")
    sid = lax.axis_index("s")
    wid = cid * 16 + sid

    ld_dst = pltpu.async_copy(dstb_hbm.at[wid], dst_v, isem)

    def sload(j, sem):
        pltpu.async_copy(srcb_hbm.at[wid, j], src_v.at[j % 2], sem)

    def slwait(j, sem):
        pltpu.make_async_copy(srcb_hbm.at[wid, j], src_v.at[j % 2], sem).wait()

    sload(0, sl0)
    sload(1, sl1)

    zv = jnp.zeros((16,), jnp.float32)

    def zrow(r, _):
        for c in range(D // 16):
            buf0[r, pl.ds(c * 16, 16)] = zv
        return 0
    lax.fori_loop(0, CHUNK, zrow, 0)

    if with_deg:
        for c in range(CHUNK // 16):
            ones_v[pl.ds(c * 16, 16)] = jnp.ones((16,), jnp.float32)

        def zdeg(i, _):
            zb_v[pl.ds(i * 16, 16)] = zv
            return 0
        lax.fori_loop(0, ROWS_PER_TILE // 16, zdeg, 0)

    zcps = [pltpu.async_copy(
        buf0, acc_sh.at[pl.ds((sid * (ROWS_PER_TILE // CHUNK) + k) * CHUNK, CHUNK), :], gsem0)
        for k in range(ROWS_PER_TILE // CHUNK)]
    if with_deg:
        zcps.append(pltpu.async_copy(
            zb_v, deg_sh.at[pl.ds(sid * ROWS_PER_TILE, ROWS_PER_TILE)], gsem0))
    for cp in zcps:
        cp.wait()
    plsc.subcore_barrier()
    ld_dst.wait()

    def gather(j, buf, sem):
        pltpu.async_copy(x_hbm.at[src_v.at[j % 2, 0]], buf, sem)

    def gwait(j, buf, sem):
        pltpu.make_async_copy(x_hbm.at[src_v.at[j % 2, 0]], buf, sem).wait()

    def scat(j, buf, sem):
        pltpu.async_copy(buf, acc_sh.at[dst_v.at[j]], sem, add=True)

    def swait(j, buf, sem):
        pltpu.make_async_copy(buf, acc_sh.at[dst_v.at[j]], sem).wait()

    def degscat(j):
        if with_deg:
            pltpu.async_copy(ones_v, deg_sh.at[dst_v.at[j]], dsem, add=True)

    slwait(0, sl0)
    gather(0, buf0, gsem0)
    slwait(1, sl1)
    gwait(0, buf0, gsem0)
    sload(2, sl0)
    gather(1, buf1, gsem1)
    scat(0, buf0, ssem0)
    degscat(0)

    def pair(k, _):
        j = 2 * k + 1
        gwait(j, buf1, gsem1)
        sload(j + 2, sl1)
        swait(j - 1, buf0, ssem0)
        slwait(j + 1, sl0)
        gather(j + 1, buf0, gsem0)
        scat(j, buf1, ssem1)
        degscat(j)
        gwait(j + 1, buf0, gsem0)
        sload(j + 3, sl0)
        swait(j, buf1, ssem1)
        slwait(j + 2, sl1)
        gather(j + 2, buf1, gsem1)
        scat(j + 1, buf0, ssem0)
        degscat(j + 1)
        return 0
    lax.fori_loop(0, (CHUNKS_PER_TILE - 3) // 2, pair, 0)

    J = CHUNKS_PER_TILE - 2
    gwait(J, buf1, gsem1)
    swait(J - 1, buf0, ssem0)
    slwait(J + 1, sl0)
    gather(J + 1, buf0, gsem0)
    scat(J, buf1, ssem1)
    degscat(J)
    gwait(J + 1, buf0, gsem0)
    swait(J, buf1, ssem1)
    scat(J + 1, buf0, ssem0)
    degscat(J + 1)
    swait(J + 1, buf0, ssem0)

    if with_deg:
        def ddrain(j, _):
            pltpu.make_async_copy(ones_v, deg_sh.at[dst_v.at[j]], dsem).wait()
            return 0
        lax.fori_loop(0, CHUNKS_PER_TILE, ddrain, 0)

    plsc.subcore_barrier()

    ocps = []
    for k in range(ROWS_PER_TILE // CHUNK):
        r0 = (sid * (ROWS_PER_TILE // CHUNK) + k) * CHUNK
        ocps.append(pltpu.async_copy(acc_sh.at[pl.ds(r0, CHUNK), :],
                                     out_hbm.at[cid, pl.ds(r0, CHUNK), :], gsem0))
    if with_deg:
        ocps.append(pltpu.async_copy(deg_sh.at[pl.ds(sid * ROWS_PER_TILE, ROWS_PER_TILE)],
                                     outdeg_hbm.at[cid, pl.ds(sid * ROWS_PER_TILE, ROWS_PER_TILE)], gsem0))
    for cp in ocps:
        cp.wait()


@functools.cache
def _get_sc_agg(with_deg):
    return pl.kernel(
        functools.partial(_sc_agg_body, with_deg),
        out_type=(jax.ShapeDtypeStruct((2, ACC_ROWS, D), jnp.float32),
                  jax.ShapeDtypeStruct((2, ACC_ROWS), jnp.float32)),
        mesh=plsc.VectorSubcoreMesh(core_axis_name="c", subcore_axis_name="s"),
        scratch_types=(
            pltpu.VMEM((2, 1, CHUNK), jnp.int32),
            pltpu.VMEM((CHUNKS_PER_TILE, CHUNK), jnp.int32),
            pltpu.VMEM((CHUNK, D), jnp.float32),
            pltpu.VMEM((CHUNK, D), jnp.float32),
            pltpu.VMEM((CHUNK,), jnp.float32),
            pltpu.VMEM((ROWS_PER_TILE,), jnp.float32),
            pltpu.VMEM_SHARED((ACC_ROWS, D), jnp.float32),
            pltpu.VMEM_SHARED((ACC_ROWS,), jnp.float32),
            pltpu.SemaphoreType.DMA,
            pltpu.SemaphoreType.DMA,
            pltpu.SemaphoreType.DMA,
            pltpu.SemaphoreType.DMA,
            pltpu.SemaphoreType.DMA,
            pltpu.SemaphoreType.DMA,
            pltpu.SemaphoreType.DMA,
            pltpu.SemaphoreType.DMA,
        ),
    )


def _tc_layer_body(x_ref, agg_ref, deg_ref, wl_ref, wr_ref, b_ref, o_ref):
    agg = agg_ref[0] + agg_ref[1]
    deg = deg_ref[0] + deg_ref[1]
    inv = jnp.reshape(1.0 / jnp.maximum(deg, 1.0), (deg.shape[0], 1))
    m = agg * inv
    dn = (((1,), (1,)), ((), ()))
    h = (lax.dot_general(m, wl_ref[...], dn, preferred_element_type=jnp.float32)
         + b_ref[...]
         + lax.dot_general(x_ref[...], wr_ref[...], dn, preferred_element_type=jnp.float32))
    o_ref[...] = 0.5 * h * (1.0 + lax.erf(h * 0.7071067811865476))


def _tc_layer(x, acc, deg, WlT, b, WrT):
    R = 2048
    grid = ACC_ROWS // R
    return pl.pallas_call(
        _tc_layer_body,
        grid=(grid,),
        in_specs=[
            pl.BlockSpec((R, D), lambda i: (i, 0)),
            pl.BlockSpec((2, R, D), lambda i: (0, i, 0)),
            pl.BlockSpec((2, R), lambda i: (0, i)),
            pl.BlockSpec((D, D), lambda i: (0, 0)),
            pl.BlockSpec((D, D), lambda i: (0, 0)),
            pl.BlockSpec((1, D), lambda i: (0, 0)),
        ],
        out_specs=pl.BlockSpec((R, D), lambda i: (i, 0)),
        out_shape=jax.ShapeDtypeStruct((N_NODES, D), jnp.float32),
    )(x, acc, deg, WlT, WrT, b.reshape(1, D))


def kernel(x, edge_index, W_l0, b_l0, W_r0, W_l1, b_l1, W_r1):
    srcb, dstb = _prep_edges(edge_index)
    acc0, deg = _get_sc_agg(True)(x, srcb, dstb)
    h0 = _tc_layer(x, acc0, deg, W_l0, b_l0, W_r0)
    acc1, _ = _get_sc_agg(False)(h0, srcb, dstb)
    return _tc_layer(h0, acc1, deg, W_l1, b_l1, W_r1)


# speedup vs baseline: 1.0040x; 1.0040x over previous
import functools

import jax
import jax.numpy as jnp
from jax import lax
from jax.experimental import pallas as pl
from jax.experimental.pallas import tpu as pltpu
from jax.experimental.pallas import tpu_sc as plsc

N_NODES = 10000
N_EDGES = 320000
D = 128

NUM_TILES = 32
CHUNK = 128
CHUNKS_PER_TILE = 79
E_PAD = NUM_TILES * CHUNKS_PER_TILE * CHUNK
ACC_ROWS = 10240
ROWS_PER_TILE = ACC_ROWS // 16


def _prep_edges(edge_index):
    ei = edge_index.astype(jnp.int32).reshape(2, N_EDGES // CHUNK, CHUNK)
    npad = E_PAD - N_EDGES
    ar = jnp.arange(npad, dtype=jnp.int32)
    pad_src = ((ar * 97) % N_NODES).reshape(npad // CHUNK, CHUNK)
    pad_dst = (N_NODES + (ar % (ACC_ROWS - N_NODES))).reshape(npad // CHUNK, CHUNK)
    srcb = jnp.concatenate([ei[0], pad_src]).reshape(NUM_TILES, CHUNKS_PER_TILE, 1, CHUNK)
    dstb = jnp.concatenate([ei[1], pad_dst]).reshape(NUM_TILES, CHUNKS_PER_TILE, CHUNK)
    return srcb, dstb


def _sc_agg_body(with_deg, x_hbm, srcb_hbm, dstb_hbm, out_hbm, outdeg_hbm,
                 src_v, dst_v, buf0, buf1, ones_v, zb_v, acc_sh, deg_sh,
                 isem, sl0, sl1, gsem0, gsem1, ssem0, ssem1, dsem):
    cid = lax.axis_index("c")
    sid = lax.axis_index("s")
    wid = cid * 16 + sid

    ld_dst = pltpu.async_copy(dstb_hbm.at[wid], dst_v, isem)

    def sload(j, sem):
        pltpu.async_copy(srcb_hbm.at[wid, j], src_v.at[j % 2], sem)

    def slwait(j, sem):
        pltpu.make_async_copy(srcb_hbm.at[wid, j], src_v.at[j % 2], sem).wait()

    sload(0, sl0)
    sload(1, sl1)

    zv = jnp.zeros((16,), jnp.float32)

    def zrow(r, _):
        for c in range(D // 16):
            buf0[r, pl.ds(c * 16, 16)] = zv
        return 0
    lax.fori_loop(0, CHUNK, zrow, 0)

    if with_deg:
        for c in range(CHUNK // 16):
            ones_v[pl.ds(c * 16, 16)] = jnp.ones((16,), jnp.float32)

        def zdeg(i, _):
            zb_v[pl.ds(i * 16, 16)] = zv
            return 0
        lax.fori_loop(0, ROWS_PER_TILE // 16, zdeg, 0)

    zcps = [pltpu.async_copy(
        buf0, acc_sh.at[pl.ds((sid * (ROWS_PER_TILE // CHUNK) + k) * CHUNK, CHUNK), :], gsem0)
        for k in range(ROWS_PER_TILE // CHUNK)]
    if with_deg:
        zcps.append(pltpu.async_copy(
            zb_v, deg_sh.at[pl.ds(sid * ROWS_PER_TILE, ROWS_PER_TILE)], gsem0))
    for cp in zcps:
        cp.wait()
    plsc.subcore_barrier()
    ld_dst.wait()

    def gather(j, buf, sem):
        pltpu.async_copy(x_hbm.at[src_v.at[j % 2, 0]], buf, sem)

    def gwait(j, buf, sem):
        pltpu.make_async_copy(x_hbm.at[src_v.at[j % 2, 0]], buf, sem).wait()

    def scat(j, buf, sem):
        pltpu.async_copy(buf, acc_sh.at[dst_v.at[j]], sem, add=True)

    def swait(j, buf, sem):
        pltpu.make_async_copy(buf, acc_sh.at[dst_v.at[j]], sem).wait()

    def degscat(j):
        if with_deg:
            pltpu.async_copy(ones_v, deg_sh.at[dst_v.at[j]], dsem, add=True)

    slwait(0, sl0)
    gather(0, buf0, gsem0)
    slwait(1, sl1)
    gwait(0, buf0, gsem0)
    sload(2, sl0)
    gather(1, buf1, gsem1)
    scat(0, buf0, ssem0)
    degscat(0)

    def pair(k, _):
        j = 2 * k + 1
        gwait(j, buf1, gsem1)
        sload(j + 2, sl1)
        swait(j - 1, buf0, ssem0)
        slwait(j + 1, sl0)
        gather(j + 1, buf0, gsem0)
        scat(j, buf1, ssem1)
        degscat(j)
        gwait(j + 1, buf0, gsem0)
        sload(j + 3, sl0)
        swait(j, buf1, ssem1)
        slwait(j + 2, sl1)
        gather(j + 2, buf1, gsem1)
        scat(j + 1, buf0, ssem0)
        degscat(j + 1)
        return 0
    lax.fori_loop(0, (CHUNKS_PER_TILE - 3) // 2, pair, 0)

    J = CHUNKS_PER_TILE - 2
    gwait(J, buf1, gsem1)
    swait(J - 1, buf0, ssem0)
    slwait(J + 1, sl0)
    gather(J + 1, buf0, gsem0)
    scat(J, buf1, ssem1)
    degscat(J)
    gwait(J + 1, buf0, gsem0)
    swait(J, buf1, ssem1)
    scat(J + 1, buf0, ssem0)
    degscat(J + 1)
    swait(J + 1, buf0, ssem0)

    if with_deg:
        def ddrain(j, _):
            pltpu.make_async_copy(ones_v, deg_sh.at[dst_v.at[j]], dsem).wait()
            return 0
        lax.fori_loop(0, CHUNKS_PER_TILE, ddrain, 0)

    plsc.subcore_barrier()

    ocps = []
    for k in range(ROWS_PER_TILE // CHUNK):
        r0 = (sid * (ROWS_PER_TILE // CHUNK) + k) * CHUNK
        ocps.append(pltpu.async_copy(acc_sh.at[pl.ds(r0, CHUNK), :],
                                     out_hbm.at[cid, pl.ds(r0, CHUNK), :], gsem0))
    if with_deg:
        ocps.append(pltpu.async_copy(deg_sh.at[pl.ds(sid * ROWS_PER_TILE, ROWS_PER_TILE)],
                                     outdeg_hbm.at[cid, pl.ds(sid * ROWS_PER_TILE, ROWS_PER_TILE)], gsem0))
    for cp in ocps:
        cp.wait()


@functools.cache
def _get_sc_agg(with_deg):
    return pl.kernel(
        functools.partial(_sc_agg_body, with_deg),
        out_type=(jax.ShapeDtypeStruct((2, ACC_ROWS, D), jnp.float32),
                  jax.ShapeDtypeStruct((2, ACC_ROWS), jnp.float32)),
        mesh=plsc.VectorSubcoreMesh(core_axis_name="c", subcore_axis_name="s"),
        scratch_types=(
            pltpu.VMEM((2, 1, CHUNK), jnp.int32),
            pltpu.VMEM((CHUNKS_PER_TILE, CHUNK), jnp.int32),
            pltpu.VMEM((CHUNK, D), jnp.float32),
            pltpu.VMEM((CHUNK, D), jnp.float32),
            pltpu.VMEM((CHUNK,), jnp.float32),
            pltpu.VMEM((ROWS_PER_TILE,), jnp.float32),
            pltpu.VMEM_SHARED((ACC_ROWS, D), jnp.float32),
            pltpu.VMEM_SHARED((ACC_ROWS,), jnp.float32),
            pltpu.SemaphoreType.DMA,
            pltpu.SemaphoreType.DMA,
            pltpu.SemaphoreType.DMA,
            pltpu.SemaphoreType.DMA,
            pltpu.SemaphoreType.DMA,
            pltpu.SemaphoreType.DMA,
            pltpu.SemaphoreType.DMA,
            pltpu.SemaphoreType.DMA,
        ),
    )


def _tc_layer_body(x_ref, agg_ref, deg_ref, wl_ref, wr_ref, b_ref, o_ref):
    agg = agg_ref[0] + agg_ref[1]
    deg = deg_ref[0] + deg_ref[1]
    inv = jnp.reshape(1.0 / jnp.maximum(deg, 1.0), (deg.shape[0], 1))
    m = agg * inv
    h = (jnp.dot(m, wl_ref[...], preferred_element_type=jnp.float32)
         + b_ref[...]
         + jnp.dot(x_ref[...], wr_ref[...], preferred_element_type=jnp.float32))
    o_ref[...] = 0.5 * h * (1.0 + lax.erf(h * 0.7071067811865476))


def _tc_layer(x, acc, deg, WlT, b, WrT):
    R = 2048
    grid = ACC_ROWS // R
    return pl.pallas_call(
        _tc_layer_body,
        grid=(grid,),
        in_specs=[
            pl.BlockSpec((R, D), lambda i: (i, 0)),
            pl.BlockSpec((2, R, D), lambda i: (0, i, 0)),
            pl.BlockSpec((2, R), lambda i: (0, i)),
            pl.BlockSpec((D, D), lambda i: (0, 0)),
            pl.BlockSpec((D, D), lambda i: (0, 0)),
            pl.BlockSpec((1, D), lambda i: (0, 0)),
        ],
        out_specs=pl.BlockSpec((R, D), lambda i: (i, 0)),
        out_shape=jax.ShapeDtypeStruct((N_NODES, D), jnp.float32),
    )(x, acc, deg, WlT, WrT, b.reshape(1, D))


def kernel(x, edge_index, W_l0, b_l0, W_r0, W_l1, b_l1, W_r1):
    srcb, dstb = _prep_edges(edge_index)
    acc0, deg = _get_sc_agg(True)(x, srcb, dstb)
    h0 = _tc_layer(x, acc0, deg, W_l0.T, b_l0, W_r0.T)
    acc1, _ = _get_sc_agg(False)(h0, srcb, dstb)
    return _tc_layer(h0, acc1, deg, W_l1.T, b_l1, W_r1.T)
